# Initial kernel scaffold; baseline (speedup 1.0000x reference)
import functools

import jax
import jax.numpy as jnp
from jax import lax
from jax.experimental import pallas as pl
from jax.experimental.pallas import tpu as pltpu
from jax.experimental.pallas import tpu_sc as plsc

N_NODES = 10000
D = 128

NC = 2
NS = 16
NW = NC * NS

CHUNK = 128
NPAD = 10240
ROWS_PER_TILE = NPAD // NS


def _sc_body(feats_hbm, src_hbm, dst_hbm, zrow_hbm, zcnt_hbm, ones_hbm,
             sums_out, cnts_out,
             sidx_v, didx_v, rows_v, ones_v, cstage_v):
  c = lax.axis_index("c")
  s = lax.axis_index("s")
  wid = c * NS + s
  n_chunks = src_hbm.shape[1]

  row0 = s * ROWS_PER_TILE

  def scoped(acc_sh, cnt_sh):
    pltpu.sync_copy(zrow_hbm, rows_v)
    for k in range(ROWS_PER_TILE // CHUNK):
      pltpu.sync_copy(rows_v, acc_sh.at[pl.ds(row0 + k * CHUNK, CHUNK)])
    pltpu.sync_copy(zcnt_hbm, cstage_v)
    pltpu.sync_copy(cstage_v, cnt_sh.at[pl.ds(row0, ROWS_PER_TILE)])
    pltpu.sync_copy(ones_hbm, ones_v)
    pltpu.sync_copy(src_hbm.at[wid], sidx_v)
    pltpu.sync_copy(dst_hbm.at[wid], didx_v)
    plsc.subcore_barrier()

    def step(j, carry):
      pltpu.sync_copy(feats_hbm.at[sidx_v.at[j]], rows_v)
      pltpu.sync_copy(rows_v, acc_sh.at[didx_v.at[j]], add=True)
      pltpu.sync_copy(ones_v, cnt_sh.at[didx_v.at[j]], add=True)
      return carry

    lax.fori_loop(0, n_chunks, step, 0)
    plsc.subcore_barrier()

    for k in range(ROWS_PER_TILE // CHUNK):
      pltpu.sync_copy(acc_sh.at[pl.ds(row0 + k * CHUNK, CHUNK)], rows_v)
      pltpu.sync_copy(rows_v, sums_out.at[c, pl.ds(row0 + k * CHUNK, CHUNK)])
    pltpu.sync_copy(cnt_sh.at[pl.ds(row0, ROWS_PER_TILE)], cstage_v)
    pltpu.sync_copy(cstage_v, cnts_out.at[c, pl.ds(row0, ROWS_PER_TILE)])

  pl.run_scoped(
      scoped,
      acc_sh=pltpu.VMEM_SHARED((NPAD, D), jnp.float32),
      cnt_sh=pltpu.VMEM_SHARED((NPAD, 16), jnp.float32),
  )


def _tc_body(sums_ref, cnts_ref, w_ref, b_ref, out_ref):
  ssum = sums_ref[0] + sums_ref[1]
  cnt = cnts_ref[0, :, 0:1] + cnts_ref[1, :, 0:1]
  neigh = ssum / jnp.maximum(cnt, 1.0)
  acc = lax.dot_general(neigh, w_ref[...], (((1,), (1,)), ((), ())),
                        preferred_element_type=jnp.float32)
  out_ref[...] = jnp.maximum(acc + b_ref[...], 0.0)


def kernel(node_feats, edge_index, W, b):
  n, d = node_feats.shape
  e = edge_index.shape[1]
  src = edge_index[0].astype(jnp.int32)
  dst = edge_index[1].astype(jnp.int32)

  e_pad = -(-e // (NW * CHUNK)) * (NW * CHUNK)
  if e_pad != e:
    pad = e_pad - e
    src = jnp.concatenate([src, jnp.zeros((pad,), jnp.int32)])
    dst = jnp.concatenate([dst, jnp.full((pad,), NPAD - 1, jnp.int32)])
  n_chunks = e_pad // (NW * CHUNK)
  src3 = src.reshape(NW, n_chunks, CHUNK)
  dst3 = dst.reshape(NW, n_chunks, CHUNK)

  zrow = jnp.zeros((CHUNK, D), jnp.float32)
  zcnt = jnp.zeros((ROWS_PER_TILE, 16), jnp.float32)
  ones = jnp.ones((CHUNK, 16), jnp.float32)

  mesh = plsc.VectorSubcoreMesh(core_axis_name="c", subcore_axis_name="s",
                                num_cores=NC, num_subcores=NS)
  sc_fn = pl.kernel(
      _sc_body,
      out_type=[
          jax.ShapeDtypeStruct((NC, NPAD, D), jnp.float32),
          jax.ShapeDtypeStruct((NC, NPAD, 16), jnp.float32),
      ],
      mesh=mesh,
      scratch_types=[
          pltpu.VMEM((n_chunks, CHUNK), jnp.int32),
          pltpu.VMEM((n_chunks, CHUNK), jnp.int32),
          pltpu.VMEM((CHUNK, D), jnp.float32),
          pltpu.VMEM((CHUNK, 16), jnp.float32),
          pltpu.VMEM((ROWS_PER_TILE, 16), jnp.float32),
      ],
  )
  sums, cnts = sc_fn(node_feats, src3, dst3, zrow, zcnt, ones)

  BR = 512
  out = pl.pallas_call(
      _tc_body,
      grid=(NPAD // BR,),
      in_specs=[
          pl.BlockSpec((NC, BR, D), lambda i: (0, i, 0)),
          pl.BlockSpec((NC, BR, 16), lambda i: (0, i, 0)),
          pl.BlockSpec((D, D), lambda i: (0, 0)),
          pl.BlockSpec((1, D), lambda i: (0, 0)),
      ],
      out_specs=pl.BlockSpec((BR, D), lambda i: (i, 0)),
      out_shape=jax.ShapeDtypeStruct((NPAD, D), jnp.float32),
  )(sums, cnts, W, b.reshape(1, D))
  return out[:n]


# speedup vs baseline: 3.3699x; 3.3699x over previous
import jax
import jax.numpy as jnp
from jax import lax
from jax.experimental import pallas as pl
from jax.experimental.pallas import tpu as pltpu
from jax.experimental.pallas import tpu_sc as plsc

D = 128

NC = 2
NS = 16
NW = NC * NS

CHUNK = 128
IB = 8
NPAD = 10240
ROWS_PER_TILE = NPAD // NS
CW = 8


def _sc_body(feats_hbm, src_hbm, dst_hbm, zrow_hbm, zcnt_hbm, ones_hbm,
             sums_out, cnts_out,
             sidx_v, didx_v, rows_v, ones_v, cstage_v, acc_sh, cnt_sh):
  c = lax.axis_index("c")
  s = lax.axis_index("s")
  wid = c * NS + s
  n_outer = src_hbm.shape[1] // IB

  row0 = s * ROWS_PER_TILE

  pltpu.sync_copy(zrow_hbm, rows_v)
  for k in range(ROWS_PER_TILE // CHUNK):
    pltpu.sync_copy(rows_v, acc_sh.at[pl.ds(row0 + k * CHUNK, CHUNK)])
  pltpu.sync_copy(zcnt_hbm, cstage_v)
  pltpu.sync_copy(cstage_v, cnt_sh.at[pl.ds(row0, ROWS_PER_TILE)])
  pltpu.sync_copy(ones_hbm, ones_v)
  plsc.subcore_barrier()

  def outer(o, carry):
    pltpu.sync_copy(src_hbm.at[wid, pl.ds(o * IB, IB)], sidx_v)
    pltpu.sync_copy(dst_hbm.at[wid, pl.ds(o * IB, IB)], didx_v)
    for b in range(IB):
      pltpu.sync_copy(feats_hbm.at[sidx_v.at[b]], rows_v)
      pltpu.sync_copy(rows_v, acc_sh.at[didx_v.at[b]], add=True)
      pltpu.sync_copy(ones_v, cnt_sh.at[didx_v.at[b]], add=True)
    return carry

  lax.fori_loop(0, n_outer, outer, 0)
  plsc.subcore_barrier()

  for k in range(ROWS_PER_TILE // CHUNK):
    pltpu.sync_copy(acc_sh.at[pl.ds(row0 + k * CHUNK, CHUNK)], rows_v)
    pltpu.sync_copy(rows_v, sums_out.at[c, pl.ds(row0 + k * CHUNK, CHUNK)])
  pltpu.sync_copy(cnt_sh.at[pl.ds(row0, ROWS_PER_TILE)], cstage_v)
  pltpu.sync_copy(cstage_v, cnts_out.at[c, pl.ds(row0, ROWS_PER_TILE)])


def _tc_body(sums_ref, cnts_ref, w_ref, b_ref, out_ref):
  ssum = sums_ref[0] + sums_ref[1]
  cnt = cnts_ref[0, :, 0:1] + cnts_ref[1, :, 0:1]
  neigh = ssum / jnp.maximum(cnt, 1.0)
  acc = lax.dot_general(neigh, w_ref[...], (((1,), (1,)), ((), ())),
                        preferred_element_type=jnp.float32)
  out_ref[...] = jnp.maximum(acc + b_ref[...], 0.0)


def kernel(node_feats, edge_index, W, b):
  n, d = node_feats.shape
  e = edge_index.shape[1]
  src = edge_index[0].astype(jnp.int32)
  dst = edge_index[1].astype(jnp.int32)

  e_pad = -(-e // (NW * IB * CHUNK)) * (NW * IB * CHUNK)
  if e_pad != e:
    pad = e_pad - e
    src = jnp.concatenate([src, jnp.zeros((pad,), jnp.int32)])
    dst = jnp.concatenate([dst, jnp.full((pad,), NPAD - 1, jnp.int32)])
  n_chunks = e_pad // (NW * CHUNK)
  src3 = src.reshape(NW, n_chunks, CHUNK)
  dst3 = dst.reshape(NW, n_chunks, CHUNK)

  zrow = jnp.zeros((CHUNK, D), jnp.float32)
  zcnt = jnp.zeros((ROWS_PER_TILE, CW), jnp.float32)
  ones = jnp.ones((CHUNK, CW), jnp.float32)

  mesh = plsc.VectorSubcoreMesh(core_axis_name="c", subcore_axis_name="s",
                                num_cores=NC, num_subcores=NS)
  sc_fn = pl.kernel(
      _sc_body,
      out_type=[
          jax.ShapeDtypeStruct((NC, NPAD, D), jnp.float32),
          jax.ShapeDtypeStruct((NC, NPAD, CW), jnp.float32),
      ],
      mesh=mesh,
      compiler_params=pltpu.CompilerParams(use_tc_tiling_on_sc=False),
      scratch_types=[
          pltpu.VMEM((IB, CHUNK), jnp.int32),
          pltpu.VMEM((IB, CHUNK), jnp.int32),
          pltpu.VMEM((CHUNK, D), jnp.float32),
          pltpu.VMEM((CHUNK, CW), jnp.float32),
          pltpu.VMEM((ROWS_PER_TILE, CW), jnp.float32),
          pltpu.VMEM_SHARED((NPAD, D), jnp.float32),
          pltpu.VMEM_SHARED((NPAD, CW), jnp.float32),
      ],
  )
  sums, cnts = sc_fn(node_feats, src3, dst3, zrow, zcnt, ones)

  BR = 512
  out = pl.pallas_call(
      _tc_body,
      grid=(NPAD // BR,),
      in_specs=[
          pl.BlockSpec((NC, BR, D), lambda i: (0, i, 0)),
          pl.BlockSpec((NC, BR, CW), lambda i: (0, i, 0)),
          pl.BlockSpec((D, D), lambda i: (0, 0)),
          pl.BlockSpec((1, D), lambda i: (0, 0)),
      ],
      out_specs=pl.BlockSpec((BR, D), lambda i: (i, 0)),
      out_shape=jax.ShapeDtypeStruct((NPAD, D), jnp.float32),
  )(sums, cnts, W, b.reshape(1, D))
  return out[:n]


# speedup vs baseline: 6.4626x; 1.9177x over previous
import jax
import jax.numpy as jnp
from jax import lax
from jax.experimental import pallas as pl
from jax.experimental.pallas import tpu as pltpu
from jax.experimental.pallas import tpu_sc as plsc

D = 128

NC = 2
NS = 16
NW = NC * NS

CHUNK = 80
NPAD = 10240
ROWS_PER_TILE = NPAD // NS
CW = 8


def _sc_body(feats_hbm, src_hbm, dst_hbm, zrow_hbm, zcnt_hbm, ones_hbm,
             sums_out, cnts_out,
             sidx_v, didx_v, rows0_v, rows1_v, ones_v, cstage_v,
             acc_sh, cnt_sh,
             sem_g0, sem_g1, sem_s0, sem_s1, sem_c0, sem_c1):
  c = lax.axis_index("c")
  s = lax.axis_index("s")
  wid = c * NS + s
  n_chunks = src_hbm.shape[1]
  n_half = n_chunks // 2

  row0 = s * ROWS_PER_TILE

  pltpu.sync_copy(zrow_hbm, rows0_v)
  for k in range(ROWS_PER_TILE // CHUNK):
    pltpu.sync_copy(rows0_v, acc_sh.at[pl.ds(row0 + k * CHUNK, CHUNK)])
  pltpu.sync_copy(zcnt_hbm, cstage_v)
  for k in range(ROWS_PER_TILE // CHUNK):
    pltpu.sync_copy(cstage_v, cnt_sh.at[pl.ds(row0 + k * CHUNK, CHUNK)])
  pltpu.sync_copy(ones_hbm, ones_v)
  pltpu.sync_copy(src_hbm.at[wid], sidx_v)
  pltpu.sync_copy(dst_hbm.at[wid], didx_v)
  plsc.subcore_barrier()

  def gather(j, rows_v, sem):
    return pltpu.async_copy(feats_hbm.at[sidx_v.at[j]], rows_v, sem)

  def scatter(j, rows_v, sem):
    return pltpu.async_copy(rows_v, acc_sh.at[didx_v.at[j]], sem, add=True)

  def counts(j, sem):
    return pltpu.async_copy(ones_v, cnt_sh.at[didx_v.at[j]], sem, add=True)

  gather(0, rows0_v, sem_g0)

  def body(i, carry):
    j0 = 2 * i
    j1 = j0 + 1
    pltpu.make_async_copy(feats_hbm.at[sidx_v.at[j0]], rows0_v, sem_g0).wait()
    scatter(j0, rows0_v, sem_s0)

    @pl.when(i > 0)
    def _():
      pltpu.make_async_copy(rows1_v, acc_sh.at[didx_v.at[j1]], sem_s1).wait()
      pltpu.make_async_copy(ones_v, cnt_sh.at[didx_v.at[j1]], sem_c1).wait()

    counts(j0, sem_c0)
    gather(j1, rows1_v, sem_g1)

    pltpu.make_async_copy(feats_hbm.at[sidx_v.at[j1]], rows1_v, sem_g1).wait()
    scatter(j1, rows1_v, sem_s1)
    pltpu.make_async_copy(rows0_v, acc_sh.at[didx_v.at[j0]], sem_s0).wait()
    pltpu.make_async_copy(ones_v, cnt_sh.at[didx_v.at[j0]], sem_c0).wait()
    counts(j1, sem_c1)

    @pl.when(i < n_half - 1)
    def _():
      gather(j0 + 2, rows0_v, sem_g0)

    return carry

  lax.fori_loop(0, n_half, body, 0)
  pltpu.make_async_copy(rows1_v, acc_sh.at[didx_v.at[0]], sem_s1).wait()
  pltpu.make_async_copy(ones_v, cnt_sh.at[didx_v.at[0]], sem_c1).wait()
  plsc.subcore_barrier()

  for k in range(ROWS_PER_TILE // CHUNK):
    pltpu.sync_copy(acc_sh.at[pl.ds(row0 + k * CHUNK, CHUNK)], rows0_v)
    pltpu.sync_copy(rows0_v, sums_out.at[c, pl.ds(row0 + k * CHUNK, CHUNK)])
    pltpu.sync_copy(cnt_sh.at[pl.ds(row0 + k * CHUNK, CHUNK)], cstage_v)
    pltpu.sync_copy(cstage_v, cnts_out.at[c, pl.ds(row0 + k * CHUNK, CHUNK)])


def _tc_body(sums_ref, cnts_ref, w_ref, b_ref, out_ref):
  ssum = sums_ref[0] + sums_ref[1]
  cnt = cnts_ref[0, :, 0:1] + cnts_ref[1, :, 0:1]
  neigh = ssum / jnp.maximum(cnt, 1.0)
  acc = lax.dot_general(neigh, w_ref[...], (((1,), (1,)), ((), ())),
                        preferred_element_type=jnp.float32)
  out_ref[...] = jnp.maximum(acc + b_ref[...], 0.0)


def kernel(node_feats, edge_index, W, b):
  n, d = node_feats.shape
  e = edge_index.shape[1]
  src = edge_index[0].astype(jnp.int32)
  dst = edge_index[1].astype(jnp.int32)

  e_pad = -(-e // (NW * 2 * CHUNK)) * (NW * 2 * CHUNK)
  if e_pad != e:
    pad = e_pad - e
    src = jnp.concatenate([src, jnp.zeros((pad,), jnp.int32)])
    dst = jnp.concatenate([dst, jnp.full((pad,), NPAD - 1, jnp.int32)])
  n_chunks = e_pad // (NW * CHUNK)
  src3 = src.reshape(NW, n_chunks, CHUNK)
  dst3 = dst.reshape(NW, n_chunks, CHUNK)

  zrow = jnp.zeros((CHUNK, D), jnp.float32)
  zcnt = jnp.zeros((CHUNK, CW), jnp.float32)
  ones = jnp.ones((CHUNK, CW), jnp.float32)

  mesh = plsc.VectorSubcoreMesh(core_axis_name="c", subcore_axis_name="s",
                                num_cores=NC, num_subcores=NS)
  sc_fn = pl.kernel(
      _sc_body,
      out_type=[
          jax.ShapeDtypeStruct((NC, NPAD, D), jnp.float32),
          jax.ShapeDtypeStruct((NC, NPAD, CW), jnp.float32),
      ],
      mesh=mesh,
      compiler_params=pltpu.CompilerParams(use_tc_tiling_on_sc=False),
      scratch_types=[
          pltpu.VMEM((n_chunks, CHUNK), jnp.int32),
          pltpu.VMEM((n_chunks, CHUNK), jnp.int32),
          pltpu.VMEM((CHUNK, D), jnp.float32),
          pltpu.VMEM((CHUNK, D), jnp.float32),
          pltpu.VMEM((CHUNK, CW), jnp.float32),
          pltpu.VMEM((CHUNK, CW), jnp.float32),
          pltpu.VMEM_SHARED((NPAD, D), jnp.float32),
          pltpu.VMEM_SHARED((NPAD, CW), jnp.float32),
          pltpu.SemaphoreType.DMA,
          pltpu.SemaphoreType.DMA,
          pltpu.SemaphoreType.DMA,
          pltpu.SemaphoreType.DMA,
          pltpu.SemaphoreType.DMA,
          pltpu.SemaphoreType.DMA,
      ],
  )
  sums, cnts = sc_fn(node_feats, src3, dst3, zrow, zcnt, ones)

  BR = 512
  out = pl.pallas_call(
      _tc_body,
      grid=(NPAD // BR,),
      in_specs=[
          pl.BlockSpec((NC, BR, D), lambda i: (0, i, 0)),
          pl.BlockSpec((NC, BR, CW), lambda i: (0, i, 0)),
          pl.BlockSpec((D, D), lambda i: (0, 0)),
          pl.BlockSpec((1, D), lambda i: (0, 0)),
      ],
      out_specs=pl.BlockSpec((BR, D), lambda i: (i, 0)),
      out_shape=jax.ShapeDtypeStruct((NPAD, D), jnp.float32),
  )(sums, cnts, W, b.reshape(1, D))
  return out[:n]


# speedup vs baseline: 6.4627x; 1.0000x over previous
import jax
import jax.numpy as jnp
from jax import lax
from jax.experimental import pallas as pl
from jax.experimental.pallas import tpu as pltpu
from jax.experimental.pallas import tpu_sc as plsc

D = 128

NC = 2
NS = 16
NW = NC * NS

CHUNK = 64
NPAD = 10240
ROWS_PER_TILE = NPAD // NS
CW = 8

SPLIT_A = 114
SPLIT_B = 200


def _sc_body(feats_hbm, src_hbm, dst_hbm, zrow_hbm, zcnt_hbm, ones_hbm,
             sums_out, cnts_out,
             sidx_v, didx_v, rows0_v, rows1_v, ones_v, cstage_v,
             acc_sh, cnt_sh,
             sem_g0, sem_g1, sem_s0, sem_s1, sem_c0, sem_c1):
  c = lax.axis_index("c")
  s = lax.axis_index("s")

  start = lax.select(c == 0, s * SPLIT_A, NS * SPLIT_A + s * SPLIT_B)
  n_half = lax.select(c == 0, SPLIT_A // 2, SPLIT_B // 2)

  row0 = s * ROWS_PER_TILE

  pltpu.sync_copy(zrow_hbm, rows0_v)
  for k in range(ROWS_PER_TILE // CHUNK):
    pltpu.sync_copy(rows0_v, acc_sh.at[pl.ds(row0 + k * CHUNK, CHUNK)])
  pltpu.sync_copy(zcnt_hbm, cstage_v)
  for k in range(ROWS_PER_TILE // CHUNK):
    pltpu.sync_copy(cstage_v, cnt_sh.at[pl.ds(row0 + k * CHUNK, CHUNK)])
  pltpu.sync_copy(ones_hbm, ones_v)
  pltpu.sync_copy(src_hbm.at[pl.ds(start, SPLIT_B)], sidx_v)
  pltpu.sync_copy(dst_hbm.at[pl.ds(start, SPLIT_B)], didx_v)
  plsc.subcore_barrier()

  def gather(j, rows_v, sem):
    return pltpu.async_copy(feats_hbm.at[sidx_v.at[j]], rows_v, sem)

  def scatter(j, rows_v, sem):
    return pltpu.async_copy(rows_v, acc_sh.at[didx_v.at[j]], sem, add=True)

  def counts(j, sem):
    return pltpu.async_copy(ones_v, cnt_sh.at[didx_v.at[j]], sem, add=True)

  gather(0, rows0_v, sem_g0)

  def body(i, carry):
    j0 = 2 * i
    j1 = j0 + 1
    pltpu.make_async_copy(feats_hbm.at[sidx_v.at[j0]], rows0_v, sem_g0).wait()
    scatter(j0, rows0_v, sem_s0)

    @pl.when(i > 0)
    def _():
      pltpu.make_async_copy(rows1_v, acc_sh.at[didx_v.at[j1]], sem_s1).wait()
      pltpu.make_async_copy(ones_v, cnt_sh.at[didx_v.at[j1]], sem_c1).wait()

    counts(j0, sem_c0)
    gather(j1, rows1_v, sem_g1)

    pltpu.make_async_copy(feats_hbm.at[sidx_v.at[j1]], rows1_v, sem_g1).wait()
    scatter(j1, rows1_v, sem_s1)
    pltpu.make_async_copy(rows0_v, acc_sh.at[didx_v.at[j0]], sem_s0).wait()
    pltpu.make_async_copy(ones_v, cnt_sh.at[didx_v.at[j0]], sem_c0).wait()
    counts(j1, sem_c1)

    @pl.when(i < n_half - 1)
    def _():
      gather(j0 + 2, rows0_v, sem_g0)

    return carry

  lax.fori_loop(0, n_half, body, 0)
  pltpu.make_async_copy(rows1_v, acc_sh.at[didx_v.at[0]], sem_s1).wait()
  pltpu.make_async_copy(ones_v, cnt_sh.at[didx_v.at[0]], sem_c1).wait()
  plsc.subcore_barrier()

  for k in range(ROWS_PER_TILE // CHUNK):
    pltpu.sync_copy(acc_sh.at[pl.ds(row0 + k * CHUNK, CHUNK)], rows0_v)
    pltpu.sync_copy(rows0_v, sums_out.at[c, pl.ds(row0 + k * CHUNK, CHUNK)])
    pltpu.sync_copy(cnt_sh.at[pl.ds(row0 + k * CHUNK, CHUNK)], cstage_v)
    pltpu.sync_copy(cstage_v, cnts_out.at[c, pl.ds(row0 + k * CHUNK, CHUNK)])


def _tc_body(sums_ref, cnts_ref, w_ref, b_ref, out_ref):
  ssum = sums_ref[0] + sums_ref[1]
  cnt = cnts_ref[0, :, 0:1] + cnts_ref[1, :, 0:1]
  neigh = ssum / jnp.maximum(cnt, 1.0)
  acc = lax.dot_general(neigh, w_ref[...], (((1,), (1,)), ((), ())),
                        preferred_element_type=jnp.float32)
  out_ref[...] = jnp.maximum(acc + b_ref[...], 0.0)


def kernel(node_feats, edge_index, W, b):
  n, d = node_feats.shape
  e = edge_index.shape[1]
  src = edge_index[0].astype(jnp.int32)
  dst = edge_index[1].astype(jnp.int32)

  tot_chunks = NS * (SPLIT_A + SPLIT_B)
  e_pad = tot_chunks * CHUNK
  assert e_pad >= e, (e_pad, e)
  if e_pad != e:
    pad = e_pad - e
    src = jnp.concatenate([src, jnp.zeros((pad,), jnp.int32)])
    dst = jnp.concatenate([dst, jnp.full((pad,), NPAD - 1, jnp.int32)])
  src2 = src.reshape(tot_chunks, CHUNK)
  dst2 = dst.reshape(tot_chunks, CHUNK)

  zrow = jnp.zeros((CHUNK, D), jnp.float32)
  zcnt = jnp.zeros((CHUNK, CW), jnp.float32)
  ones = jnp.ones((CHUNK, CW), jnp.float32)

  mesh = plsc.VectorSubcoreMesh(core_axis_name="c", subcore_axis_name="s",
                                num_cores=NC, num_subcores=NS)
  sc_fn = pl.kernel(
      _sc_body,
      out_type=[
          jax.ShapeDtypeStruct((NC, NPAD, D), jnp.float32),
          jax.ShapeDtypeStruct((NC, NPAD, CW), jnp.float32),
      ],
      mesh=mesh,
      compiler_params=pltpu.CompilerParams(use_tc_tiling_on_sc=False),
      scratch_types=[
          pltpu.VMEM((SPLIT_B, CHUNK), jnp.int32),
          pltpu.VMEM((SPLIT_B, CHUNK), jnp.int32),
          pltpu.VMEM((CHUNK, D), jnp.float32),
          pltpu.VMEM((CHUNK, D), jnp.float32),
          pltpu.VMEM((CHUNK, CW), jnp.float32),
          pltpu.VMEM((CHUNK, CW), jnp.float32),
          pltpu.VMEM_SHARED((NPAD, D), jnp.float32),
          pltpu.VMEM_SHARED((NPAD, CW), jnp.float32),
          pltpu.SemaphoreType.DMA,
          pltpu.SemaphoreType.DMA,
          pltpu.SemaphoreType.DMA,
          pltpu.SemaphoreType.DMA,
          pltpu.SemaphoreType.DMA,
          pltpu.SemaphoreType.DMA,
      ],
  )
  sums, cnts = sc_fn(node_feats, src2, dst2, zrow, zcnt, ones)

  BR = 512
  out = pl.pallas_call(
      _tc_body,
      grid=(NPAD // BR,),
      in_specs=[
          pl.BlockSpec((NC, BR, D), lambda i: (0, i, 0)),
          pl.BlockSpec((NC, BR, CW), lambda i: (0, i, 0)),
          pl.BlockSpec((D, D), lambda i: (0, 0)),
          pl.BlockSpec((1, D), lambda i: (0, 0)),
      ],
      out_specs=pl.BlockSpec((BR, D), lambda i: (i, 0)),
      out_shape=jax.ShapeDtypeStruct((NPAD, D), jnp.float32),
  )(sums, cnts, W, b.reshape(1, D))
  return out[:n]


# speedup vs baseline: 7.5092x; 1.1619x over previous
import jax
import jax.numpy as jnp
from jax import lax
from jax.experimental import pallas as pl
from jax.experimental.pallas import tpu as pltpu
from jax.experimental.pallas import tpu_sc as plsc

D = 128

NC = 2
NS = 16
NW = NC * NS

CHUNK = 64
NPAD = 10240
ROWS_PER_TILE = NPAD // NS
CW = 8

SPLIT_A = 200
SPLIT_B = 114
MAXSPLIT = max(SPLIT_A, SPLIT_B)


def _sc_body(feats_hbm, src_hbm, dst_hbm, zrow_hbm, zcnt_hbm, ones_hbm,
             sums_out, cnts_out,
             sidx_v, didx_v, rows0_v, rows1_v, ones_v, cstage_v,
             acc_sh, cnt_sh,
             sem_g0, sem_g1, sem_s0, sem_s1, sem_c0, sem_c1):
  c = lax.axis_index("c")
  s = lax.axis_index("s")

  start = lax.select(c == 0, s * SPLIT_A, NS * SPLIT_A + s * SPLIT_B)
  n_half = lax.select(c == 0, SPLIT_A // 2, SPLIT_B // 2)

  row0 = s * ROWS_PER_TILE

  pltpu.sync_copy(zrow_hbm, rows0_v)
  for k in range(ROWS_PER_TILE // CHUNK):
    pltpu.sync_copy(rows0_v, acc_sh.at[pl.ds(row0 + k * CHUNK, CHUNK)])
  pltpu.sync_copy(zcnt_hbm, cstage_v)
  for k in range(ROWS_PER_TILE // CHUNK):
    pltpu.sync_copy(cstage_v, cnt_sh.at[pl.ds(row0 + k * CHUNK, CHUNK)])
  pltpu.sync_copy(ones_hbm, ones_v)
  pltpu.sync_copy(src_hbm.at[pl.ds(start, MAXSPLIT)], sidx_v)
  pltpu.sync_copy(dst_hbm.at[pl.ds(start, MAXSPLIT)], didx_v)
  plsc.subcore_barrier()

  def gather(j, rows_v, sem):
    return pltpu.async_copy(feats_hbm.at[sidx_v.at[j]], rows_v, sem)

  def scatter(j, rows_v, sem):
    return pltpu.async_copy(rows_v, acc_sh.at[didx_v.at[j]], sem, add=True)

  def counts(j, sem):
    return pltpu.async_copy(ones_v, cnt_sh.at[didx_v.at[j]], sem, add=True)

  gather(0, rows0_v, sem_g0)

  def body(i, carry):
    j0 = 2 * i
    j1 = j0 + 1
    pltpu.make_async_copy(feats_hbm.at[sidx_v.at[j0]], rows0_v, sem_g0).wait()
    scatter(j0, rows0_v, sem_s0)

    @pl.when(i > 0)
    def _():
      pltpu.make_async_copy(rows1_v, acc_sh.at[didx_v.at[j1]], sem_s1).wait()
      pltpu.make_async_copy(ones_v, cnt_sh.at[didx_v.at[j1]], sem_c1).wait()

    counts(j0, sem_c0)
    gather(j1, rows1_v, sem_g1)

    pltpu.make_async_copy(feats_hbm.at[sidx_v.at[j1]], rows1_v, sem_g1).wait()
    scatter(j1, rows1_v, sem_s1)
    pltpu.make_async_copy(rows0_v, acc_sh.at[didx_v.at[j0]], sem_s0).wait()
    pltpu.make_async_copy(ones_v, cnt_sh.at[didx_v.at[j0]], sem_c0).wait()
    counts(j1, sem_c1)

    @pl.when(i < n_half - 1)
    def _():
      gather(j0 + 2, rows0_v, sem_g0)

    return carry

  lax.fori_loop(0, n_half, body, 0)
  pltpu.make_async_copy(rows1_v, acc_sh.at[didx_v.at[0]], sem_s1).wait()
  pltpu.make_async_copy(ones_v, cnt_sh.at[didx_v.at[0]], sem_c1).wait()
  plsc.subcore_barrier()

  for k in range(ROWS_PER_TILE // CHUNK):
    pltpu.sync_copy(acc_sh.at[pl.ds(row0 + k * CHUNK, CHUNK)], rows0_v)
    pltpu.sync_copy(rows0_v, sums_out.at[c, pl.ds(row0 + k * CHUNK, CHUNK)])
    pltpu.sync_copy(cnt_sh.at[pl.ds(row0 + k * CHUNK, CHUNK)], cstage_v)
    pltpu.sync_copy(cstage_v, cnts_out.at[c, pl.ds(row0 + k * CHUNK, CHUNK)])


def _tc_body(sums_ref, cnts_ref, w_ref, b_ref, out_ref):
  ssum = sums_ref[0] + sums_ref[1]
  cnt = cnts_ref[0, :, 0:1] + cnts_ref[1, :, 0:1]
  neigh = ssum / jnp.maximum(cnt, 1.0)
  acc = lax.dot_general(neigh, w_ref[...], (((1,), (1,)), ((), ())),
                        preferred_element_type=jnp.float32)
  out_ref[...] = jnp.maximum(acc + b_ref[...], 0.0)


def kernel(node_feats, edge_index, W, b):
  n, d = node_feats.shape
  e = edge_index.shape[1]
  src = edge_index[0].astype(jnp.int32)
  dst = edge_index[1].astype(jnp.int32)

  tot_chunks = NS * (SPLIT_A + SPLIT_B)
  e_pad = tot_chunks * CHUNK
  assert e_pad >= e, (e_pad, e)
  if e_pad != e:
    pad = e_pad - e
    src = jnp.concatenate([src, jnp.zeros((pad,), jnp.int32)])
    dst = jnp.concatenate([dst, jnp.full((pad,), NPAD - 1, jnp.int32)])
  src2 = src.reshape(tot_chunks, CHUNK)
  dst2 = dst.reshape(tot_chunks, CHUNK)
  extra = MAXSPLIT - SPLIT_B
  if extra:
    src2 = jnp.concatenate([src2, jnp.zeros((extra, CHUNK), jnp.int32)])
    dst2 = jnp.concatenate(
        [dst2, jnp.full((extra, CHUNK), NPAD - 1, jnp.int32)])

  zrow = jnp.zeros((CHUNK, D), jnp.float32)
  zcnt = jnp.zeros((CHUNK, CW), jnp.float32)
  ones = jnp.ones((CHUNK, CW), jnp.float32)

  mesh = plsc.VectorSubcoreMesh(core_axis_name="c", subcore_axis_name="s",
                                num_cores=NC, num_subcores=NS)
  sc_fn = pl.kernel(
      _sc_body,
      out_type=[
          jax.ShapeDtypeStruct((NC, NPAD, D), jnp.float32),
          jax.ShapeDtypeStruct((NC, NPAD, CW), jnp.float32),
      ],
      mesh=mesh,
      compiler_params=pltpu.CompilerParams(use_tc_tiling_on_sc=False),
      scratch_types=[
          pltpu.VMEM((MAXSPLIT, CHUNK), jnp.int32),
          pltpu.VMEM((MAXSPLIT, CHUNK), jnp.int32),
          pltpu.VMEM((CHUNK, D), jnp.float32),
          pltpu.VMEM((CHUNK, D), jnp.float32),
          pltpu.VMEM((CHUNK, CW), jnp.float32),
          pltpu.VMEM((CHUNK, CW), jnp.float32),
          pltpu.VMEM_SHARED((NPAD, D), jnp.float32),
          pltpu.VMEM_SHARED((NPAD, CW), jnp.float32),
          pltpu.SemaphoreType.DMA,
          pltpu.SemaphoreType.DMA,
          pltpu.SemaphoreType.DMA,
          pltpu.SemaphoreType.DMA,
          pltpu.SemaphoreType.DMA,
          pltpu.SemaphoreType.DMA,
      ],
  )
  sums, cnts = sc_fn(node_feats, src2, dst2, zrow, zcnt, ones)

  BR = 512
  out = pl.pallas_call(
      _tc_body,
      grid=(NPAD // BR,),
      in_specs=[
          pl.BlockSpec((NC, BR, D), lambda i: (0, i, 0)),
          pl.BlockSpec((NC, BR, CW), lambda i: (0, i, 0)),
          pl.BlockSpec((D, D), lambda i: (0, 0)),
          pl.BlockSpec((1, D), lambda i: (0, 0)),
      ],
      out_specs=pl.BlockSpec((BR, D), lambda i: (i, 0)),
      out_shape=jax.ShapeDtypeStruct((NPAD, D), jnp.float32),
  )(sums, cnts, W, b.reshape(1, D))
  return out[:n]


# speedup vs baseline: 7.5119x; 1.0004x over previous
import jax
import jax.numpy as jnp
from jax import lax
from jax.experimental import pallas as pl
from jax.experimental.pallas import tpu as pltpu
from jax.experimental.pallas import tpu_sc as plsc

D = 128

NC = 2
NS = 16
NW = NC * NS

CHUNK = 64
NPAD = 10240
ROWS_PER_TILE = NPAD // NS
CW = 8

SPLIT_A = 200
SPLIT_B = 114
MAXSPLIT = max(SPLIT_A, SPLIT_B)


def _sc_body(feats_hbm, src_hbm, dst_hbm, zrow_hbm, zcnt_hbm, ones_hbm,
             sums_out, cnts_out,
             sidx_v, didx_v, rows0_v, rows1_v, ones_v, cstage_v,
             acc_sh, cnt_sh,
             sem_g0, sem_g1, sem_s0, sem_s1, sem_c0, sem_c1):
  c = lax.axis_index("c")
  s = lax.axis_index("s")

  start = lax.select(c == 0, s * SPLIT_A, NS * SPLIT_A + s * SPLIT_B)
  n_half = lax.select(c == 0, SPLIT_A // 2, SPLIT_B // 2)

  row0 = s * ROWS_PER_TILE

  pltpu.sync_copy(zrow_hbm, rows0_v)
  for k in range(ROWS_PER_TILE // CHUNK):
    pltpu.sync_copy(rows0_v, acc_sh.at[pl.ds(row0 + k * CHUNK, CHUNK)])
  pltpu.sync_copy(zcnt_hbm, cstage_v)
  for k in range(ROWS_PER_TILE // CHUNK):
    pltpu.sync_copy(cstage_v, cnt_sh.at[pl.ds(row0 + k * CHUNK, CHUNK)])
  pltpu.sync_copy(ones_hbm, ones_v)
  pltpu.sync_copy(src_hbm.at[pl.ds(start, MAXSPLIT)], sidx_v)
  pltpu.sync_copy(dst_hbm.at[pl.ds(start, MAXSPLIT)], didx_v)
  plsc.subcore_barrier()

  H = CHUNK // 2

  def gather(j, rows_v, sem):
    pltpu.async_copy(feats_hbm.at[sidx_v.at[j, pl.ds(0, H)]],
                     rows_v.at[pl.ds(0, H)], sem)
    pltpu.async_copy(feats_hbm.at[sidx_v.at[j, pl.ds(H, H)]],
                     rows_v.at[pl.ds(H, H)], sem)

  def gather_wait(j, rows_v, sem):
    pltpu.make_async_copy(feats_hbm.at[sidx_v.at[j, pl.ds(0, H)]],
                          rows_v.at[pl.ds(0, H)], sem).wait()
    pltpu.make_async_copy(feats_hbm.at[sidx_v.at[j, pl.ds(H, H)]],
                          rows_v.at[pl.ds(H, H)], sem).wait()

  def scatter(j, rows_v, sem):
    return pltpu.async_copy(rows_v, acc_sh.at[didx_v.at[j]], sem, add=True)

  def counts(j, sem):
    return pltpu.async_copy(ones_v, cnt_sh.at[didx_v.at[j]], sem, add=True)

  gather(0, rows0_v, sem_g0)

  def body(i, carry):
    j0 = 2 * i
    j1 = j0 + 1
    gather_wait(j0, rows0_v, sem_g0)
    scatter(j0, rows0_v, sem_s0)

    @pl.when(i > 0)
    def _():
      pltpu.make_async_copy(rows1_v, acc_sh.at[didx_v.at[j1]], sem_s1).wait()
      pltpu.make_async_copy(ones_v, cnt_sh.at[didx_v.at[j1]], sem_c1).wait()

    counts(j0, sem_c0)
    gather(j1, rows1_v, sem_g1)

    gather_wait(j1, rows1_v, sem_g1)
    scatter(j1, rows1_v, sem_s1)
    pltpu.make_async_copy(rows0_v, acc_sh.at[didx_v.at[j0]], sem_s0).wait()
    pltpu.make_async_copy(ones_v, cnt_sh.at[didx_v.at[j0]], sem_c0).wait()
    counts(j1, sem_c1)

    @pl.when(i < n_half - 1)
    def _():
      gather(j0 + 2, rows0_v, sem_g0)

    return carry

  lax.fori_loop(0, n_half, body, 0)
  pltpu.make_async_copy(rows1_v, acc_sh.at[didx_v.at[0]], sem_s1).wait()
  pltpu.make_async_copy(ones_v, cnt_sh.at[didx_v.at[0]], sem_c1).wait()
  plsc.subcore_barrier()

  for k in range(ROWS_PER_TILE // CHUNK):
    pltpu.sync_copy(acc_sh.at[pl.ds(row0 + k * CHUNK, CHUNK)], rows0_v)
    pltpu.sync_copy(rows0_v, sums_out.at[c, pl.ds(row0 + k * CHUNK, CHUNK)])
    pltpu.sync_copy(cnt_sh.at[pl.ds(row0 + k * CHUNK, CHUNK)], cstage_v)
    pltpu.sync_copy(cstage_v, cnts_out.at[c, pl.ds(row0 + k * CHUNK, CHUNK)])


def _tc_body(sums_ref, cnts_ref, w_ref, b_ref, out_ref):
  ssum = sums_ref[0] + sums_ref[1]
  cnt = cnts_ref[0, :, 0:1] + cnts_ref[1, :, 0:1]
  neigh = ssum / jnp.maximum(cnt, 1.0)
  acc = lax.dot_general(neigh, w_ref[...], (((1,), (1,)), ((), ())),
                        preferred_element_type=jnp.float32)
  out_ref[...] = jnp.maximum(acc + b_ref[...], 0.0)


def kernel(node_feats, edge_index, W, b):
  n, d = node_feats.shape
  e = edge_index.shape[1]
  src = edge_index[0].astype(jnp.int32)
  dst = edge_index[1].astype(jnp.int32)

  tot_chunks = NS * (SPLIT_A + SPLIT_B)
  e_pad = tot_chunks * CHUNK
  assert e_pad >= e, (e_pad, e)
  if e_pad != e:
    pad = e_pad - e
    src = jnp.concatenate([src, jnp.zeros((pad,), jnp.int32)])
    dst = jnp.concatenate([dst, jnp.full((pad,), NPAD - 1, jnp.int32)])
  src2 = src.reshape(tot_chunks, CHUNK)
  dst2 = dst.reshape(tot_chunks, CHUNK)
  extra = MAXSPLIT - SPLIT_B
  if extra:
    src2 = jnp.concatenate([src2, jnp.zeros((extra, CHUNK), jnp.int32)])
    dst2 = jnp.concatenate(
        [dst2, jnp.full((extra, CHUNK), NPAD - 1, jnp.int32)])

  zrow = jnp.zeros((CHUNK, D), jnp.float32)
  zcnt = jnp.zeros((CHUNK, CW), jnp.float32)
  ones = jnp.ones((CHUNK, CW), jnp.float32)

  mesh = plsc.VectorSubcoreMesh(core_axis_name="c", subcore_axis_name="s",
                                num_cores=NC, num_subcores=NS)
  sc_fn = pl.kernel(
      _sc_body,
      out_type=[
          jax.ShapeDtypeStruct((NC, NPAD, D), jnp.float32),
          jax.ShapeDtypeStruct((NC, NPAD, CW), jnp.float32),
      ],
      mesh=mesh,
      compiler_params=pltpu.CompilerParams(use_tc_tiling_on_sc=False),
      scratch_types=[
          pltpu.VMEM((MAXSPLIT, CHUNK), jnp.int32),
          pltpu.VMEM((MAXSPLIT, CHUNK), jnp.int32),
          pltpu.VMEM((CHUNK, D), jnp.float32),
          pltpu.VMEM((CHUNK, D), jnp.float32),
          pltpu.VMEM((CHUNK, CW), jnp.float32),
          pltpu.VMEM((CHUNK, CW), jnp.float32),
          pltpu.VMEM_SHARED((NPAD, D), jnp.float32),
          pltpu.VMEM_SHARED((NPAD, CW), jnp.float32),
          pltpu.SemaphoreType.DMA,
          pltpu.SemaphoreType.DMA,
          pltpu.SemaphoreType.DMA,
          pltpu.SemaphoreType.DMA,
          pltpu.SemaphoreType.DMA,
          pltpu.SemaphoreType.DMA,
      ],
  )
  sums, cnts = sc_fn(node_feats, src2, dst2, zrow, zcnt, ones)

  BR = 512
  out = pl.pallas_call(
      _tc_body,
      grid=(NPAD // BR,),
      in_specs=[
          pl.BlockSpec((NC, BR, D), lambda i: (0, i, 0)),
          pl.BlockSpec((NC, BR, CW), lambda i: (0, i, 0)),
          pl.BlockSpec((D, D), lambda i: (0, 0)),
          pl.BlockSpec((1, D), lambda i: (0, 0)),
      ],
      out_specs=pl.BlockSpec((BR, D), lambda i: (i, 0)),
      out_shape=jax.ShapeDtypeStruct((NPAD, D), jnp.float32),
  )(sums, cnts, W, b.reshape(1, D))
  return out[:n]
